# Initial kernel scaffold; baseline (speedup 1.0000x reference)
import jax
import jax.numpy as jnp
from jax.experimental import pallas as pl


def kernel(x, x_sim_list, edge_index, batch, root_n_id, enc_w0, enc_b0, enc_w, enc_b, ctrl_w0, ctrl_b0, ctrl_w, ctrl_b, cond_w, cond_b, adapt_w, adapt_b, zero_w, zero_b, cls_w, cls_b):
    raise NotImplementedError("write your pallas kernel here")


# speedup vs baseline: 2.9765x; 2.9765x over previous
import functools

import jax
import jax.numpy as jnp
from jax import lax
from jax.experimental import pallas as pl
from jax.experimental.pallas import tpu as pltpu
from jax.experimental.pallas import tpu_sc as plsc

N = 10000
E = 320000
L = 5
POS = 32
H = 128
NIN = POS + 1
C = 40
G = 256
RESIDUAL_SCALE = 0.01

NTILES = 16
KCHUNK = 128
EPAD = 321536
EPT = EPAD // NTILES
NCHUNK = EPT // KCHUNK
NPAD = 10240
ROWS_PT = NPAD // NTILES

BLK = 2000


def _sc_agg_body(srcs_hbm, dst_hbm, table_hbm, zeros_hbm, out_hbm,
                 agg_sh, src_v, dst_v, rows_v, sem):
    c = lax.axis_index("c")
    s = lax.axis_index("s")
    pltpu.sync_copy(zeros_hbm, rows_v)
    for j in range(ROWS_PT // KCHUNK):
        pltpu.sync_copy(
            rows_v, agg_sh.at[pl.ds(s * ROWS_PT + j * KCHUNK, KCHUNK)])
    plsc.subcore_barrier()
    base = s * EPT

    def body(j, carry):
        off = base + j * KCHUNK
        pltpu.sync_copy(srcs_hbm.at[pl.ds(c * EPAD + off, KCHUNK)], src_v)
        pltpu.sync_copy(dst_hbm.at[pl.ds(off, KCHUNK)], dst_v)
        pltpu.async_copy(table_hbm.at[src_v], rows_v, sem).wait()
        pltpu.sync_copy(rows_v, agg_sh.at[dst_v], add=True)
        return carry

    lax.fori_loop(0, NCHUNK, body, 0)
    plsc.subcore_barrier()
    pltpu.sync_copy(agg_sh.at[pl.ds(s * ROWS_PT, ROWS_PT)],
                    out_hbm.at[pl.ds(c * NPAD + s * ROWS_PT, ROWS_PT)])


@functools.lru_cache(maxsize=1)
def _sc_agg_kernel():
    return pl.kernel(
        _sc_agg_body,
        out_type=jax.ShapeDtypeStruct((2 * NPAD, H), jnp.float32),
        mesh=plsc.VectorSubcoreMesh(
            core_axis_name="c", subcore_axis_name="s",
            num_cores=2, num_subcores=NTILES),
        scratch_types=[
            pltpu.VMEM_SHARED((NPAD, H), jnp.float32),
            pltpu.VMEM((KCHUNK,), jnp.int32),
            pltpu.VMEM((KCHUNK,), jnp.int32),
            pltpu.VMEM((KCHUNK, H), jnp.float32),
            pltpu.SemaphoreType.DMA,
        ],
    )


def _agg_call(srcs, dstp, table, zeros_blk):
    out = _sc_agg_kernel()(srcs, dstp, table, zeros_blk)
    return out.reshape(2, NPAD, H)


def _prep_body(x_ref, xs0_ref, root_ref, cw_ref, cb_ref, aw_ref, ab_ref,
               tab_ref):
    i = pl.program_id(0)
    xb = x_ref[...]
    rows = lax.broadcasted_iota(jnp.int32, (BLK, G), 0) + i * BLK
    m = (rows == root_ref[...]).astype(jnp.float32)
    seed = jnp.minimum(jnp.sum(m, axis=1, keepdims=True), 1.0)
    h0 = jnp.concatenate(
        [xb, seed, jnp.zeros((BLK, H - NIN), jnp.float32)], axis=1)
    cond0 = jnp.dot(xs0_ref[...], cw_ref[...],
                    preferred_element_type=jnp.float32) + cb_ref[...]
    cfirst = jnp.dot(cond0, aw_ref[...],
                     preferred_element_type=jnp.float32) + ab_ref[...]
    tab_ref[0] = h0
    tab_ref[1] = h0 + cfirst


def _prep_call(x, xs0, root2d, cond_w, cond_b2d, aw, ab):
    full = lambda i: (0, 0)
    return pl.pallas_call(
        _prep_body,
        grid=(N // BLK,),
        in_specs=[
            pl.BlockSpec((BLK, POS), lambda i: (i, 0)),
            pl.BlockSpec((BLK, POS), lambda i: (i, 0)),
            pl.BlockSpec((1, G), full),
            pl.BlockSpec((POS, H), full),
            pl.BlockSpec((1, H), full),
            pl.BlockSpec((H, H), full),
            pl.BlockSpec((1, H), full),
        ],
        out_specs=pl.BlockSpec((2, BLK, H), lambda i: (0, i, 0)),
        out_shape=jax.ShapeDtypeStruct((2, N, H), jnp.float32),
    )(x, xs0, root2d, cond_w, cond_b2d, aw, ab)


def _layer_body(tab_ref, agg_ref, xsn_ref, wf_ref, bf_ref, wc_ref, bc_ref,
                zw_ref, zb_ref, cw_ref, cb_ref, acc_ref,
                tabn_ref, accn_ref):
    hf = jnp.maximum(
        jnp.dot(tab_ref[0] + agg_ref[0], wf_ref[...],
                preferred_element_type=jnp.float32) + bf_ref[...], 0.0)
    hc = jnp.maximum(
        jnp.dot(tab_ref[1] + agg_ref[1], wc_ref[...],
                preferred_element_type=jnp.float32) + bc_ref[...], 0.0)
    z = jnp.dot(hc, zw_ref[...], preferred_element_type=jnp.float32) \
        + zb_ref[...]
    hf_new = hf + RESIDUAL_SCALE * z
    condn = jnp.dot(xsn_ref[...], cw_ref[...],
                    preferred_element_type=jnp.float32) + cb_ref[...]
    tabn_ref[0] = hf_new
    tabn_ref[1] = hc + condn
    accn_ref[...] = acc_ref[...] + hf_new


def _layer_call(tab, agg, xsn, wf, bf, wc, bc, zw, zb, cond_w, cond_b2d, acc):
    full = lambda i: (0, 0)
    return pl.pallas_call(
        _layer_body,
        grid=(N // BLK,),
        in_specs=[
            pl.BlockSpec((2, BLK, H), lambda i: (0, i, 0)),
            pl.BlockSpec((2, BLK, H), lambda i: (0, i, 0)),
            pl.BlockSpec((BLK, POS), lambda i: (i, 0)),
            pl.BlockSpec((H, H), full),
            pl.BlockSpec((1, H), full),
            pl.BlockSpec((H, H), full),
            pl.BlockSpec((1, H), full),
            pl.BlockSpec((H, H), full),
            pl.BlockSpec((1, H), full),
            pl.BlockSpec((POS, H), full),
            pl.BlockSpec((1, H), full),
            pl.BlockSpec((BLK, H), lambda i: (i, 0)),
        ],
        out_specs=[
            pl.BlockSpec((2, BLK, H), lambda i: (0, i, 0)),
            pl.BlockSpec((BLK, H), lambda i: (i, 0)),
        ],
        out_shape=[
            jax.ShapeDtypeStruct((2, N, H), jnp.float32),
            jax.ShapeDtypeStruct((N, H), jnp.float32),
        ],
    )(tab, agg, xsn, wf, bf, wc, bc, zw, zb, cond_w, cond_b2d, acc)


def _pool_body(acc_ref, batch_ref, clsw_ref, clsb_ref, out_ref):
    oh = (lax.broadcasted_iota(jnp.int32, (G, N), 0)
          == batch_ref[...]).astype(jnp.float32)
    pooled = jnp.dot(oh, acc_ref[...], preferred_element_type=jnp.float32)
    cnt = jnp.sum(oh, axis=1, keepdims=True)
    pooled = pooled / jnp.maximum(cnt, 1.0)
    nrm = jnp.sqrt(jnp.sum(pooled * pooled, axis=1, keepdims=True))
    pooled = pooled / jnp.maximum(nrm, 1e-5)
    out_ref[...] = jnp.dot(pooled, clsw_ref[...],
                           preferred_element_type=jnp.float32) + clsb_ref[...]


def _pool_call(acc, batch2d, clsw_pad, clsb_pad):
    return pl.pallas_call(
        _pool_body,
        out_shape=jax.ShapeDtypeStruct((G, H), jnp.float32),
    )(acc, batch2d, clsw_pad, clsb_pad)


def kernel(x, x_sim_list, edge_index, batch, root_n_id, enc_w0, enc_b0,
           enc_w, enc_b, ctrl_w0, ctrl_b0, ctrl_w, ctrl_b, cond_w, cond_b,
           adapt_w, adapt_b, zero_w, zero_b, cls_w, cls_b):
    f32 = jnp.float32
    src = jnp.pad(edge_index[0], (0, EPAD - E))
    srcs = jnp.concatenate([src, src + N])
    dstp = jnp.pad(edge_index[1], (0, EPAD - E),
                   constant_values=N)
    zeros_blk = jnp.zeros((KCHUNK, H), f32)

    wf0 = jnp.pad(enc_w0, ((0, H - NIN), (0, 0)))
    wc0 = jnp.pad(ctrl_w0, ((0, H - NIN), (0, 0)))
    aw = jnp.pad(adapt_w, ((0, 0), (0, H - NIN)))
    ab = jnp.pad(adapt_b, (0, H - NIN))[None]
    cond_b2d = cond_b[None]

    tab = _prep_call(x, x_sim_list[0], root_n_id[None].astype(jnp.int32),
                     cond_w, cond_b2d, aw, ab)
    acc = jnp.zeros((N, H), f32)
    for i in range(L):
        agg = _agg_call(srcs, dstp, tab.reshape(2 * N, H), zeros_blk)
        wf = wf0 if i == 0 else enc_w[i - 1]
        bf = (enc_b0 if i == 0 else enc_b[i - 1])[None]
        wc = wc0 if i == 0 else ctrl_w[i - 1]
        bc = (ctrl_b0 if i == 0 else ctrl_b[i - 1])[None]
        xsn = x_sim_list[(i + 1) % L]
        tab, acc = _layer_call(tab, agg, xsn, wf, bf, wc, bc,
                               zero_w[i], zero_b[i][None], cond_w, cond_b2d,
                               acc)
    out = _pool_call(acc, batch[None], jnp.pad(cls_w, ((0, 0), (0, H - C))),
                     jnp.pad(cls_b, (0, H - C))[None])
    return out[:, :C]
